# baseline (speedup 1.0000x reference)
import functools

import jax
import jax.numpy as jnp
from jax import lax
from jax.experimental import pallas as pl
from jax.experimental.pallas import tpu as pltpu
from jax.experimental.pallas import tpu_sc as plsc

_NC = 2
_NS = 16
_NW = _NC * _NS
_C = 128
_NBUF = 6
_K = 3


def _emb_body(n_rows, t_len, seq_hbm, table_hbm, out_hbm, idx2, idx_v, *scratch):
    rows = scratch[:_NBUF]
    gs = scratch[_NBUF:2 * _NBUF]
    ws = scratch[2 * _NBUF:]

    wid = lax.axis_index("s") * _NC + lax.axis_index("c")
    per_w = n_rows // _NW
    n_chunks = per_w // _C
    wbase = wid * per_w
    seq_rows_w = per_w // t_len

    pltpu.sync_copy(
        seq_hbm.at[pl.ds(pl.multiple_of(wid * seq_rows_w, 8), seq_rows_w)], idx2)

    offs = list(range(0, t_len - 15, 16))
    if t_len % 16:
        offs.append(t_len - 16)

    def flat_row(r, carry):
        base = r * t_len
        for o in offs:
            idx_v[pl.ds(pl.multiple_of(base + o, 8), 16)] = idx2[r, pl.ds(o, 16)]
        return carry

    lax.fori_loop(0, seq_rows_w, flat_row, 0)

    def fire_g(j, b):
        off = pl.multiple_of(j * _C, _C)
        pltpu.async_copy(table_hbm.at[idx_v.at[pl.ds(off, _C)]], rows[b], gs[b])

    def wait_g(b):
        pltpu.make_async_copy(table_hbm.at[pl.ds(0, _C)], rows[b], gs[b]).wait()

    def fire_wb(j, b):
        off = pl.multiple_of(wbase + j * _C, _C)
        pltpu.async_copy(rows[b], out_hbm.at[pl.ds(off, _C)], ws[b])

    def wait_wb(b):
        pltpu.make_async_copy(table_hbm.at[pl.ds(0, _C)], rows[b], ws[b]).wait()

    wb_fired = [0] * _NBUF
    wb_waited = [0] * _NBUF

    for j in range(_K):
        fire_g(j, j % _NBUF)
    for j in range(_NBUF - _K):
        b = j % _NBUF
        wait_g(b)
        fire_wb(j, b)
        wb_fired[b] += 1
        fire_g(j + _K, (j + _K) % _NBUF)

    start = _NBUF - _K
    n_main = ((n_chunks - _K) - start) // _NBUF

    def outer(g, carry):
        for u in range(_NBUF):
            j = _NBUF * g + u + start
            b = (u + start) % _NBUF
            bk = (u + start + _K) % _NBUF
            wait_g(b)
            fire_wb(j, b)
            wait_wb(bk)
            fire_g(j + _K, bk)
        return carry

    lax.fori_loop(0, n_main, outer, 0)
    for u in range(_NBUF):
        wb_fired[(u + start) % _NBUF] += n_main
        wb_waited[(u + start + _K) % _NBUF] += n_main

    for j in range(start + n_main * _NBUF, n_chunks - _K):
        b = j % _NBUF
        bk = (j + _K) % _NBUF
        wait_g(b)
        fire_wb(j, b)
        wb_fired[b] += 1
        wait_wb(bk)
        wb_waited[bk] += 1
        fire_g(j + _K, bk)

    for j in range(n_chunks - _K, n_chunks):
        b = j % _NBUF
        wait_g(b)
        fire_wb(j, b)
        wb_fired[b] += 1

    for b in range(_NBUF):
        for _ in range(wb_fired[b] - wb_waited[b]):
            wait_wb(b)


def kernel(seq, table):
    B, T = seq.shape
    V, D = table.shape
    n = B * T

    mesh = plsc.VectorSubcoreMesh(core_axis_name="c", subcore_axis_name="s")
    run = pl.kernel(
        functools.partial(_emb_body, n, T),
        mesh=mesh,
        out_type=jax.ShapeDtypeStruct((n, D), jnp.float32),
        scratch_types=(
            [pltpu.VMEM((B // _NW, T), jnp.int32),
             pltpu.VMEM((n // _NW,), jnp.int32)]
            + [pltpu.VMEM((_C, D), jnp.float32) for _ in range(_NBUF)]
            + [pltpu.SemaphoreType.DMA for _ in range(2 * _NBUF)]
        ),
    )
    out = run(seq.astype(jnp.int32), table)
    return out.reshape(B, T, D)


# speedup vs baseline: 1.0047x; 1.0047x over previous
import functools

import jax
import jax.numpy as jnp
from jax import lax
from jax.experimental import pallas as pl
from jax.experimental.pallas import tpu as pltpu
from jax.experimental.pallas import tpu_sc as plsc

_NC = 2
_NS = 16
_NW = _NC * _NS
_C = 128
_NBUF = 6
_K = 3


def _emb_body(n_rows, t_len, seq_hbm, table_hbm, out_hbm, idx2, idx_v, *scratch):
    rows = scratch[:_NBUF]
    gs = scratch[_NBUF:2 * _NBUF]
    ws = scratch[2 * _NBUF:]

    wid = lax.axis_index("s") * _NC + lax.axis_index("c")
    per_w = n_rows // _NW
    n_chunks = per_w // _C
    wbase = wid * per_w
    seq_rows_w = per_w // t_len

    pltpu.sync_copy(
        seq_hbm.at[pl.ds(pl.multiple_of(wid * seq_rows_w, 8), seq_rows_w)], idx2)

    offs = list(range(0, t_len - 15, 16))
    if t_len % 16:
        offs.append(t_len - 16)

    def flat_row(r):
        base = r * t_len
        for o in offs:
            idx_v[pl.ds(base + o, 16)] = idx2[r, pl.ds(o, 16)]

    head_rows = -(-(_K * _C) // t_len)
    for r in range(head_rows):
        flat_row(r)

    def fire_g(j, b):
        off = pl.multiple_of(j * _C, _C)
        pltpu.async_copy(table_hbm.at[idx_v.at[pl.ds(off, _C)]], rows[b], gs[b])

    def wait_g(b):
        pltpu.make_async_copy(table_hbm.at[pl.ds(0, _C)], rows[b], gs[b]).wait()

    def fire_wb(j, b):
        off = pl.multiple_of(wbase + j * _C, _C)
        pltpu.async_copy(rows[b], out_hbm.at[pl.ds(off, _C)], ws[b])

    def wait_wb(b):
        pltpu.make_async_copy(table_hbm.at[pl.ds(0, _C)], rows[b], ws[b]).wait()

    wb_fired = [0] * _NBUF
    wb_waited = [0] * _NBUF

    for j in range(_K):
        fire_g(j, j % _NBUF)
    for r in range(head_rows, seq_rows_w):
        flat_row(r)
    for j in range(_NBUF - _K):
        b = j % _NBUF
        wait_g(b)
        fire_wb(j, b)
        wb_fired[b] += 1
        fire_g(j + _K, (j + _K) % _NBUF)

    start = _NBUF - _K
    n_main = ((n_chunks - _K) - start) // _NBUF

    def outer(g, carry):
        for u in range(_NBUF):
            j = _NBUF * g + u + start
            b = (u + start) % _NBUF
            bk = (u + start + _K) % _NBUF
            wait_g(b)
            fire_wb(j, b)
            wait_wb(bk)
            fire_g(j + _K, bk)
        return carry

    lax.fori_loop(0, n_main, outer, 0)
    for u in range(_NBUF):
        wb_fired[(u + start) % _NBUF] += n_main
        wb_waited[(u + start + _K) % _NBUF] += n_main

    for j in range(start + n_main * _NBUF, n_chunks - _K):
        b = j % _NBUF
        bk = (j + _K) % _NBUF
        wait_g(b)
        fire_wb(j, b)
        wb_fired[b] += 1
        wait_wb(bk)
        wb_waited[bk] += 1
        fire_g(j + _K, bk)

    for j in range(n_chunks - _K, n_chunks):
        b = j % _NBUF
        wait_g(b)
        fire_wb(j, b)
        wb_fired[b] += 1

    for b in range(_NBUF):
        for _ in range(wb_fired[b] - wb_waited[b]):
            wait_wb(b)


def kernel(seq, table):
    B, T = seq.shape
    V, D = table.shape
    n = B * T

    mesh = plsc.VectorSubcoreMesh(core_axis_name="c", subcore_axis_name="s")
    run = pl.kernel(
        functools.partial(_emb_body, n, T),
        mesh=mesh,
        out_type=jax.ShapeDtypeStruct((n, D), jnp.float32),
        scratch_types=(
            [pltpu.VMEM((B // _NW, T), jnp.int32),
             pltpu.VMEM((n // _NW,), jnp.int32)]
            + [pltpu.VMEM((_C, D), jnp.float32) for _ in range(_NBUF)]
            + [pltpu.SemaphoreType.DMA for _ in range(2 * _NBUF)]
        ),
    )
    out = run(seq.astype(jnp.int32), table)
    return out.reshape(B, T, D)
